# baseline (speedup 1.0000x reference)
import functools

import jax
import jax.numpy as jnp
from jax import lax
from jax.experimental import pallas as pl
from jax.experimental.pallas import tpu as pltpu
from jax.experimental.pallas import tpu_sc as plsc

VOCAB = 1000000
NQ = 100
NY = 14
D = 16
B = 16384
PAD = 101

NC = 2
NS = 16
L = 16
NW = NC * NS
N = B * NQ
ROWS_W = N // NW
CH = 512
NCHUNK = ROWS_W // CH
BPW = B // NW
NCOMBO = 2 * NY * NQ


def _sc_body(ans_hbm, year_hbm, table_hbm, yt_hbm, qt_hbm, alpha_hbm, beta_hbm,
             out_hbm,
             ans_v, cidx_v, rows_v, ybase_v, yt_v, qt_v, ayt_v, pad_v,
             alpha_v, beta_v, combo_v, gsem):
    wid = lax.axis_index("s") * NC + lax.axis_index("c")
    row0 = wid * ROWS_W
    b0 = wid * BPW

    iota = lax.iota(jnp.int32, L)

    pltpu.sync_copy(year_hbm.at[pl.ds(b0, BPW)], ybase_v)
    pltpu.sync_copy(yt_hbm, yt_v)
    pltpu.sync_copy(qt_hbm, qt_v)
    pltpu.sync_copy(table_hbm.at[pl.ds(96, 8), :], pad_v)
    pltpu.sync_copy(alpha_hbm, alpha_v)
    pltpu.sync_copy(beta_hbm, beta_v)

    @pl.loop(0, BPW // L)
    def _(k):
        s = pl.ds(k * L, L)
        ybase_v[s] = ybase_v[s] * NQ

    alpha = alpha_v[...]
    beta = beta_v[...]
    padrow = pad_v[PAD % 8, :]

    for y in range(NY):
        ayt_v[y, :] = alpha * yt_v[y, :]

    @pl.loop(0, NQ)
    def _(q):
        bq = beta * qt_v[q, :]
        for y in range(NY):
            r0 = bq + ayt_v[y, :]
            combo_v[pl.ds((y * NQ + q) * D, D)] = r0
            combo_v[pl.ds((NY * NQ + y * NQ + q) * D, D)] = r0 - padrow

    @pl.loop(0, NCHUNK)
    def _(c):
        rowbase = pl.multiple_of(row0 + c * CH, CH)

        r128 = pl.multiple_of(rowbase // 128, CH // 128)
        pltpu.sync_copy(ans_hbm.at[pl.ds(r128, CH // 128), :], ans_v)

        descs = []
        for j in range(CH // 128):
            descs.append(
                pltpu.async_copy(table_hbm.at[ans_v.at[j]],
                                 rows_v.at[pl.ds(j * 128, 128), :], gsem))

        @pl.loop(0, CH // L)
        def _(k):
            n = rowbase + k * L + iota
            b = ((n.astype(jnp.float32) + 0.5) * (1.0 / NQ)).astype(jnp.int32)
            q = n - b * NQ
            ybase = plsc.load_gather(ybase_v, [b - b0])
            av = ans_v[k // (128 // L), pl.ds((k % (128 // L)) * L, L)]
            bank = jnp.where(av == PAD, NY * NQ, 0)
            cidx_v[pl.ds(k * L, L)] = ybase + q + bank

        for d in descs:
            d.wait()

        @pl.loop(0, CH // L)
        def _(g):
            cbase = cidx_v[pl.ds(g * L, L)] * D
            rowvec = g * L + iota
            for d in range(D):
                col = plsc.load_gather(combo_v, [cbase + d])
                plsc.addupdate_scatter(rows_v, [rowvec, jnp.full((L,), d, jnp.int32)], col)

        pltpu.sync_copy(rows_v, out_hbm.at[pl.ds(rowbase, CH), :])


@jax.jit
def _run(ans2d, year_i32, answer_table, year_table, question_table,
         alpha16, beta16):
    mesh = plsc.VectorSubcoreMesh(core_axis_name="c", subcore_axis_name="s",
                                  num_cores=NC, num_subcores=NS)
    fn = pl.kernel(
        _sc_body,
        out_type=jax.ShapeDtypeStruct((N, D), jnp.float32),
        mesh=mesh,
        compiler_params=pltpu.CompilerParams(use_tc_tiling_on_sc=False,
                                             needs_layout_passes=False),
        scratch_types=[
            pltpu.VMEM((CH // 128, 128), jnp.int32),
            pltpu.VMEM((CH,), jnp.int32),
            pltpu.VMEM((CH, D), jnp.float32),
            pltpu.VMEM((BPW,), jnp.int32),
            pltpu.VMEM((NY, D), jnp.float32),
            pltpu.VMEM((NQ, D), jnp.float32),
            pltpu.VMEM((NY, D), jnp.float32),
            pltpu.VMEM((8, D), jnp.float32),
            pltpu.VMEM((L,), jnp.float32),
            pltpu.VMEM((L,), jnp.float32),
            pltpu.VMEM((NCOMBO * D,), jnp.float32),
            pltpu.SemaphoreType.DMA,
        ],
    )
    return fn(ans2d, year_i32, answer_table, year_table, question_table,
              alpha16, beta16)


def kernel(year, answer, answer_table, year_table, question_table, alpha, beta):
    ans2d = answer.astype(jnp.int32).reshape(N // 128, 128)
    year_i32 = year.astype(jnp.int32)
    alpha16 = jnp.broadcast_to(alpha.astype(jnp.float32), (L,))
    beta16 = jnp.broadcast_to(beta.astype(jnp.float32), (L,))
    out = _run(ans2d, year_i32, answer_table, year_table, question_table,
               alpha16, beta16)
    return out.reshape(B, NQ, D)


# speedup vs baseline: 1.0254x; 1.0254x over previous
import jax
import jax.numpy as jnp
from jax import lax
from jax.experimental import pallas as pl
from jax.experimental.pallas import tpu as pltpu
from jax.experimental.pallas import tpu_sc as plsc

VOCAB = 1000000
NQ = 100
NY = 14
D = 16
B = 16384
PAD = 101

NC = 2
NS = 16
L = 16
NW = NC * NS
N = B * NQ
ROWS_W = N // NW
CH = 512
NCHUNK = ROWS_W // CH
BPW = B // NW
NCOMBO = 2 * NY * NQ
NG = CH // 128


def _sc_body(ans_hbm, year_hbm, table_hbm, yt_hbm, qt_hbm, alpha_hbm, beta_hbm,
             out_hbm,
             ans_a, ans_b, cidx_a, cidx_b, rows_a, rows_b, ybase_v, combo_v,
             gsem_a, gsem_b, ssem):
    wid = lax.axis_index("s") * NC + lax.axis_index("c")
    row0 = wid * ROWS_W
    b0 = wid * BPW

    iota = lax.iota(jnp.int32, L)
    ans_bufs = (ans_a, ans_b)
    cidx_bufs = (cidx_a, cidx_b)
    rows_bufs = (rows_a, rows_b)
    gsems = (gsem_a, gsem_b)

    pltpu.sync_copy(year_hbm.at[pl.ds(b0, BPW)], ybase_v)
    yt_v = rows_a.at[pl.ds(0, NY), :]
    qt_v = rows_a.at[pl.ds(NY, NQ), :]
    pad_v = rows_a.at[pl.ds(120, 8), :]
    ab_v = rows_a.at[pl.ds(114, 2), :]
    pltpu.sync_copy(yt_hbm, yt_v)
    pltpu.sync_copy(qt_hbm, qt_v)
    pltpu.sync_copy(table_hbm.at[pl.ds(96, 8), :], pad_v)
    pltpu.sync_copy(alpha_hbm, ab_v.at[0, :])
    pltpu.sync_copy(beta_hbm, ab_v.at[1, :])

    @pl.loop(0, BPW // L)
    def _(k):
        s = pl.ds(k * L, L)
        ybase_v[s] = ybase_v[s] * NQ

    alpha = ab_v[0, :]
    beta = ab_v[1, :]
    padrow = pad_v[PAD % 8, :]

    @pl.loop(0, NQ)
    def _(q):
        bq = beta * qt_v[q, :]
        for y in range(NY):
            r0 = bq + alpha * yt_v[y, :]
            combo_v[pl.ds((y * NQ + q) * D, D)] = r0
            combo_v[pl.ds((NY * NQ + y * NQ + q) * D, D)] = r0 - padrow

    def stage_and_fire(c, par):
        rowbase = pl.multiple_of(row0 + c * CH, CH)
        pltpu.sync_copy(ans_hbm.at[pl.ds(rowbase, CH)], ans_bufs[par])
        for j in range(NG):
            pltpu.async_copy(table_hbm.at[ans_bufs[par].at[pl.ds(j * 128, 128)]],
                             rows_bufs[par].at[pl.ds(j * 128, 128), :],
                             gsems[par])

    def compute_cidx(c, par):
        rowbase = pl.multiple_of(row0 + c * CH, CH)
        cidx_v = cidx_bufs[par]
        ans_v = ans_bufs[par]

        @pl.loop(0, CH // L)
        def _(k):
            n = rowbase + k * L + iota
            b = ((n.astype(jnp.float32) + 0.5) * (1.0 / NQ)).astype(jnp.int32)
            q = n - b * NQ
            ybase = plsc.load_gather(ybase_v, [b - b0])
            av = ans_v[pl.ds(k * L, L)]
            bank = jnp.where(av == PAD, NY * NQ, 0)
            cidx_v[pl.ds(k * L, L)] = ybase + q + bank

    def wait_gathers(par):
        for j in range(NG):
            pltpu.make_async_copy(
                table_hbm.at[ans_bufs[par].at[pl.ds(j * 128, 128)]],
                rows_bufs[par].at[pl.ds(j * 128, 128), :],
                gsems[par]).wait()

    def add_combo(par):
        cidx_v = cidx_bufs[par]
        rows_v = rows_bufs[par]

        @pl.loop(0, CH // L)
        def _(g):
            cbase = cidx_v[pl.ds(g * L, L)] * D
            rowvec = g * L + iota
            for d in range(D):
                col = plsc.load_gather(combo_v, [cbase + d])
                plsc.addupdate_scatter(
                    rows_v, [rowvec, jnp.full((L,), d, jnp.int32)], col)

    def fire_scatter(c, par):
        rowbase = pl.multiple_of(row0 + c * CH, CH)
        pltpu.async_copy(rows_bufs[par], out_hbm.at[pl.ds(rowbase, CH), :],
                         ssem)

    def wait_scatter(c, par):
        rowbase = pl.multiple_of(row0 + c * CH, CH)
        pltpu.make_async_copy(rows_bufs[par],
                              out_hbm.at[pl.ds(rowbase, CH), :], ssem).wait()

    stage_and_fire(0, 0)

    @pl.loop(0, NCHUNK // 2)
    def _(c2):
        for par in range(2):
            c = c2 * 2 + par
            @pl.when(c > 0)
            def _():
                wait_scatter(c - 1, 1 - par)

            @pl.when(c + 1 < NCHUNK)
            def _():
                stage_and_fire(c + 1, 1 - par)

            compute_cidx(c, par)
            wait_gathers(par)
            add_combo(par)
            fire_scatter(c, par)

    wait_scatter(NCHUNK - 1, 1)


@jax.jit
def _run(ans_flat, year_i32, answer_table, year_table, question_table,
         alpha16, beta16):
    mesh = plsc.VectorSubcoreMesh(core_axis_name="c", subcore_axis_name="s",
                                  num_cores=NC, num_subcores=NS)
    fn = pl.kernel(
        _sc_body,
        out_type=jax.ShapeDtypeStruct((N, D), jnp.float32),
        mesh=mesh,
        compiler_params=pltpu.CompilerParams(use_tc_tiling_on_sc=False,
                                             needs_layout_passes=False),
        scratch_types=[
            pltpu.VMEM((CH,), jnp.int32),
            pltpu.VMEM((CH,), jnp.int32),
            pltpu.VMEM((CH,), jnp.int32),
            pltpu.VMEM((CH,), jnp.int32),
            pltpu.VMEM((CH, D), jnp.float32),
            pltpu.VMEM((CH, D), jnp.float32),
            pltpu.VMEM((BPW,), jnp.int32),
            pltpu.VMEM((NCOMBO * D,), jnp.float32),
            pltpu.SemaphoreType.DMA,
            pltpu.SemaphoreType.DMA,
            pltpu.SemaphoreType.DMA,
        ],
    )
    return fn(ans_flat, year_i32, answer_table, year_table, question_table,
              alpha16, beta16)


def kernel(year, answer, answer_table, year_table, question_table, alpha, beta):
    ans_flat = answer.astype(jnp.int32).reshape(N)
    year_i32 = year.astype(jnp.int32)
    alpha16 = jnp.broadcast_to(alpha.astype(jnp.float32), (L,))
    beta16 = jnp.broadcast_to(beta.astype(jnp.float32), (L,))
    out = _run(ans_flat, year_i32, answer_table, year_table, question_table,
               alpha16, beta16)
    return out.reshape(B, NQ, D)
